# baseline (speedup 1.0000x reference)
import jax
import jax.numpy as jnp
from jax import lax
from jax.experimental import pallas as pl
from jax.experimental.pallas import tpu as pltpu
from jax.experimental.pallas import tpu_sc as plsc

N_ROWS = 4096
N_COLS = 4096
NNZ = 167772

L = 16
NC, NS = 2, 16
NW = NC * NS
OUT_ELEMS = N_ROWS * N_COLS
W_ELEMS = OUT_ELEMS // NW
CAPW = 8192
WIN = CAPW + L
SENT = 0x7FFFFFFF
NNZ_SRT = 176000
NSUB = 16
SUB = W_ELEMS // NSUB
NB = 48


def _sc_body(k_hbm, v_hbm, b_hbm, out_hbm,
             kwin, vwin, cb0, cb1, bnd, sem_w, sem_d):
    wid = lax.axis_index("s") * NC + lax.axis_index("c")
    lo = wid * W_ELEMS

    cpb = pltpu.async_copy(b_hbm, bnd, sem_w)
    zeros = jnp.zeros((L,), jnp.float32)

    cpb.wait()
    bv = bnd[pl.ds(wid, L)]
    s0 = bv[0]
    s = pl.multiple_of(jnp.bitwise_and(s0, -8), 8)
    e = bv[1]
    ng = jnp.minimum((e - s + (L - 1)) // L, CAPW // L)

    cpk = pltpu.async_copy(k_hbm.at[pl.ds(s, WIN)], kwin, sem_w)
    cpv = pltpu.async_copy(v_hbm.at[pl.ds(s, WIN)], vwin, sem_w)
    cpk.wait()
    cpv.wait()

    cps = []
    for sub in range(NSUB):
        buf = cb0 if sub % 2 == 0 else cb1
        if sub >= 2:
            cps[sub - 2].wait()

        def zbody(i, c, buf=buf):
            for u in range(8):
                buf[pl.ds((i * 8 + u) * L, L)] = zeros
            return c

        lax.fori_loop(0, SUB // L // 8, zbody, 0)

        base = lo + sub * SUB

        def group(g, c, buf=buf, base=base):
            kc = kwin[pl.ds(g * L, L)]
            kn = kwin[pl.ds(g * L + 1, L)]
            vv = vwin[pl.ds(g * L, L)]
            idxl = kc - base
            insub = idxl.astype(jnp.uint32) < jnp.uint32(SUB)
            alive = jnp.logical_and(insub, kc != kn)
            plsc.store_scatter(buf, [idxl], vv, mask=alive)
            return c

        lax.fori_loop(0, ng, group, 0)

        cps.append(
            pltpu.async_copy(buf, out_hbm.at[pl.ds(base, SUB)], sem_d)
        )

    cps[NSUB - 2].wait()
    cps[NSUB - 1].wait()


@jax.jit
def _to_dense(k, v, bnd):
    kern = pl.kernel(
        _sc_body,
        out_type=jax.ShapeDtypeStruct((OUT_ELEMS,), jnp.float32),
        mesh=plsc.VectorSubcoreMesh(core_axis_name="c", subcore_axis_name="s"),
        compiler_params=pltpu.CompilerParams(needs_layout_passes=False),
        scratch_types=[
            pltpu.VMEM((WIN,), jnp.int32),
            pltpu.VMEM((WIN,), jnp.float32),
            pltpu.VMEM((SUB,), jnp.float32),
            pltpu.VMEM((SUB,), jnp.float32),
            pltpu.VMEM((NB,), jnp.int32),
            pltpu.SemaphoreType.DMA,
            pltpu.SemaphoreType.DMA,
        ],
    )
    return kern(k, v, bnd)


def kernel(indices, values):
    idx = indices.astype(jnp.int32)
    lin = idx[:, 0] * N_COLS + idx[:, 1]
    k, v = lax.sort((lin, values), num_keys=1, is_stable=False)
    padn = NNZ_SRT - NNZ
    k = jnp.concatenate([k, jnp.full((padn,), SENT, jnp.int32)])
    v = jnp.concatenate([v, jnp.zeros((padn,), jnp.float32)])
    bnd = jnp.searchsorted(
        k, jnp.arange(NW + 1, dtype=jnp.int32) * W_ELEMS, side="left"
    ).astype(jnp.int32)
    bnd = jnp.concatenate([bnd, jnp.zeros((NB - NW - 1,), jnp.int32)])
    out = _to_dense(k, v, bnd)
    return out.reshape(N_ROWS, N_COLS)


# speedup vs baseline: 1.2943x; 1.2943x over previous
import jax
import jax.numpy as jnp
from jax import lax
from jax.experimental import pallas as pl
from jax.experimental.pallas import tpu as pltpu
from jax.experimental.pallas import tpu_sc as plsc

N_ROWS = 4096
N_COLS = 4096
NNZ = 167772

L = 16
NC, NS = 2, 16
NW = NC * NS
OUT_ELEMS = N_ROWS * N_COLS
W_ELEMS = OUT_ELEMS // NW
CAPW = 8192
WIN = CAPW + L
SENT = 0x7FFFFFFF
NNZ_SRT = 176000
NSUB = 16
SUB = W_ELEMS // NSUB
NB = 48


def _sc_body(k_hbm, v_hbm, b_hbm, out_hbm,
             kwin, vwin, cb0, cb1, bnd, sem_w, sem_d):
    wid = lax.axis_index("s") * NC + lax.axis_index("c")
    lo = wid * W_ELEMS

    cpb = pltpu.async_copy(b_hbm, bnd, sem_w)
    zeros = jnp.zeros((L,), jnp.float32)

    cpb.wait()
    bv = bnd[pl.ds(wid, L)]
    s0 = bv[0]
    s = pl.multiple_of(jnp.bitwise_and(s0, -8), 8)
    e = bv[1]
    ng = jnp.minimum((e - s + (L - 1)) // L, CAPW // L)

    cpk = pltpu.async_copy(k_hbm.at[pl.ds(s, WIN)], kwin, sem_w)
    cpv = pltpu.async_copy(v_hbm.at[pl.ds(s, WIN)], vwin, sem_w)
    cpk.wait()
    cpv.wait()

    cps = []
    for sub in range(NSUB):
        buf = cb0 if sub % 2 == 0 else cb1
        if sub >= 2:
            cps[sub - 2].wait()

        def zbody(i, c, buf=buf):
            r = i // (N_COLS // L // 8)
            j = i % (N_COLS // L // 8)
            for u in range(8):
                buf[r, pl.ds((j * 8 + u) * L, L)] = zeros
            return c

        lax.fori_loop(0, SUB // L // 8, zbody, 0)

        base = lo + sub * SUB

        def group(g, c, buf=buf, base=base):
            kc = kwin[pl.ds(g * L, L)]
            kn = kwin[pl.ds(g * L + 1, L)]
            vv = vwin[pl.ds(g * L, L)]
            idxl = kc - base
            insub = idxl.astype(jnp.uint32) < jnp.uint32(SUB)
            alive = jnp.logical_and(insub, kc != kn)
            ir = lax.shift_right_logical(idxl, 12)
            ic = jnp.bitwise_and(idxl, N_COLS - 1)
            plsc.store_scatter(buf, [ir, ic], vv, mask=alive)
            return c

        lax.fori_loop(0, ng, group, 0)

        row0 = wid * (N_ROWS // NW) + sub * 8
        cps.append(
            pltpu.async_copy(buf, out_hbm.at[pl.ds(row0, 8), :], sem_d)
        )

    cps[NSUB - 2].wait()
    cps[NSUB - 1].wait()


@jax.jit
def _to_dense(k, v, bnd):
    kern = pl.kernel(
        _sc_body,
        out_type=jax.ShapeDtypeStruct((N_ROWS, N_COLS), jnp.float32),
        mesh=plsc.VectorSubcoreMesh(core_axis_name="c", subcore_axis_name="s"),
        compiler_params=pltpu.CompilerParams(needs_layout_passes=False),
        scratch_types=[
            pltpu.VMEM((WIN,), jnp.int32),
            pltpu.VMEM((WIN,), jnp.float32),
            pltpu.VMEM((8, N_COLS), jnp.float32),
            pltpu.VMEM((8, N_COLS), jnp.float32),
            pltpu.VMEM((NB,), jnp.int32),
            pltpu.SemaphoreType.DMA,
            pltpu.SemaphoreType.DMA,
        ],
    )
    return kern(k, v, bnd)


def kernel(indices, values):
    idx = indices.astype(jnp.int32)
    lin = idx[:, 0] * N_COLS + idx[:, 1]
    k, v = lax.sort((lin, values), num_keys=1, is_stable=False)
    padn = NNZ_SRT - NNZ
    k = jnp.concatenate([k, jnp.full((padn,), SENT, jnp.int32)])
    v = jnp.concatenate([v, jnp.zeros((padn,), jnp.float32)])
    bnd = jnp.searchsorted(
        k, jnp.arange(NW + 1, dtype=jnp.int32) * W_ELEMS, side="left"
    ).astype(jnp.int32)
    bnd = jnp.concatenate([bnd, jnp.zeros((NB - NW - 1,), jnp.int32)])
    return _to_dense(k, v, bnd)
